# baseline (speedup 1.0000x reference)
import functools

import numpy as np
import jax
import jax.numpy as jnp
from jax.experimental import pallas as pl
from jax.sharding import Mesh, PartitionSpec as P

_N = 10000
_D = 128
_BM = 200


def _bdot(a, b):
    return jnp.dot(a.astype(jnp.bfloat16), b.astype(jnp.bfloat16),
                   preferred_element_type=jnp.float32)


def _gru_update(h, m, wu_h, wu_m, wr_h, wr_m, wc, bu, br):
    z = jax.nn.sigmoid(_bdot(h, wu_h) + _bdot(m, wu_m) + bu)
    r = jax.nn.sigmoid(_bdot(h, wr_h) + _bdot(m, wr_m) + br)
    cand = jnp.tanh(_bdot(r * h, wc))
    return z * h + (1.0 - z) * cand


def _h0_kernel(x_ref, w_ref, b_ref, h32_ref, h16_ref):
    h = _bdot(x_ref[...], w_ref[...]) + b_ref[...]
    h32_ref[...] = h
    h16_ref[...] = h.astype(jnp.bfloat16)


def _step_kernel(adj_ref, hb_ref, h_ref, wu_h_ref, wu_m_ref, wr_h_ref,
                 wr_m_ref, wc_ref, bu_ref, br_ref, nh32_ref, nh16_ref,
                 *maybe_adj16_ref, cast_adj):
    adj = adj_ref[...].astype(jnp.bfloat16)
    if cast_adj:
        maybe_adj16_ref[0][...] = adj
    m = jnp.dot(adj, hb_ref[...], preferred_element_type=jnp.float32)
    h_new = _gru_update(h_ref[...], m, wu_h_ref[...], wu_m_ref[...],
                        wr_h_ref[...], wr_m_ref[...], wc_ref[...],
                        bu_ref[...], br_ref[...])
    nh32_ref[...] = h_new
    nh16_ref[...] = h_new.astype(jnp.bfloat16)


def _row_spec(bm, width):
    return pl.BlockSpec((bm, width), lambda i: (i, 0))


def _full_spec(shape):
    return pl.BlockSpec(shape, lambda i: (0, 0))


def _local_pipeline(rows, gather, input, adj, weight, bias, candidate_weight,
                    wu_h, wu_m, wr_h, wr_m, bu, br):
    h_shapes = [jax.ShapeDtypeStruct((rows, _D), jnp.float32),
                jax.ShapeDtypeStruct((rows, _D), jnp.bfloat16)]

    bm0 = 1000 if rows % 1000 == 0 else _BM
    h32, h16l = pl.pallas_call(
        _h0_kernel,
        grid=(rows // bm0,),
        in_specs=[_row_spec(bm0, _D), _full_spec((_D, _D)),
                  _full_spec((1, _D))],
        out_specs=[_row_spec(bm0, _D), _row_spec(bm0, _D)],
        out_shape=h_shapes,
    )(input, weight, bias)

    small_specs = [
        _full_spec((_D, _D)), _full_spec((_D, _D)), _full_spec((_D, _D)),
        _full_spec((_D, _D)), _full_spec((_D, _D)), _full_spec((1, _D)),
        _full_spec((1, _D)),
    ]
    small_args = (wu_h, wu_m, wr_h, wr_m, candidate_weight, bu, br)
    h_out_specs = [_row_spec(_BM, _D), _row_spec(_BM, _D)]

    h16 = gather(h16l)

    h32, h16l, adj16 = pl.pallas_call(
        functools.partial(_step_kernel, cast_adj=True),
        grid=(rows // _BM,),
        in_specs=[_row_spec(_BM, _N), _full_spec((_N, _D)),
                  _row_spec(_BM, _D)] + small_specs,
        out_specs=h_out_specs + [_row_spec(_BM, _N)],
        out_shape=h_shapes + [jax.ShapeDtypeStruct((rows, _N), jnp.bfloat16)],
    )(adj, h16, h32, *small_args)
    h16 = gather(h16l)

    step = pl.pallas_call(
        functools.partial(_step_kernel, cast_adj=False),
        grid=(rows // _BM,),
        in_specs=[_row_spec(_BM, _N), _full_spec((_N, _D)),
                  _row_spec(_BM, _D)] + small_specs,
        out_specs=h_out_specs,
        out_shape=h_shapes,
    )
    h32, h16l = step(adj16, h16, h32, *small_args)
    h16 = gather(h16l)
    h32, _ = step(adj16, h16, h32, *small_args)
    return h32


def kernel(input, adj, weight, bias, candidate_weight, update_w, update_b,
           reset_w, reset_b):
    wu_h, wu_m = update_w[:_D], update_w[_D:]
    wr_h, wr_m = reset_w[:_D], reset_w[_D:]
    bu = update_b.reshape(1, _D)
    br = reset_b.reshape(1, _D)
    bias2 = bias.reshape(1, _D)

    devs = jax.devices()
    n_shards = 2 if len(devs) >= 2 and _N % (2 * _BM) == 0 else 1
    if n_shards == 1:
        return _local_pipeline(_N, lambda x: x, input, adj, weight, bias2,
                               candidate_weight, wu_h, wu_m, wr_h, wr_m,
                               bu, br)

    mesh = Mesh(np.array(devs[:n_shards]), ("x",))
    rows = _N // n_shards

    def gather(x):
        return jax.lax.all_gather(x, "x", tiled=True)

    def sharded(input, adj, weight, bias2, candidate_weight,
                wu_h, wu_m, wr_h, wr_m, bu, br):
        h32 = _local_pipeline(rows, gather, input, adj, weight, bias2,
                              candidate_weight, wu_h, wu_m, wr_h, wr_m,
                              bu, br)
        return jax.lax.all_gather(h32, "x", tiled=True)

    rep = P(None, None)
    return jax.shard_map(
        sharded, mesh=mesh,
        in_specs=(P("x", None), P("x", None), rep, rep, rep, rep, rep,
                  rep, rep, rep, rep),
        out_specs=rep, check_vma=False,
    )(input, adj, weight, bias2, candidate_weight,
      wu_h, wu_m, wr_h, wr_m, bu, br)


# speedup vs baseline: 2.4977x; 2.4977x over previous
import functools

import jax
import jax.numpy as jnp
from jax.experimental import pallas as pl

_N = 10000
_D = 128
_BM1 = 400
_BM2 = 400


def _bdot(a, b):
    return jnp.dot(a.astype(jnp.bfloat16), b.astype(jnp.bfloat16),
                   preferred_element_type=jnp.float32)


def _gru_update(h, m, w_gate, wc, b_gate):
    gi = jnp.concatenate([h, m], axis=1)
    pre = _bdot(gi, w_gate) + b_gate
    z = jax.nn.sigmoid(pre[:, :_D])
    r = jax.nn.sigmoid(pre[:, _D:])
    cand = jnp.tanh(_bdot(r * h, wc))
    return z * h + (1.0 - z) * cand


def _h0_kernel(x_ref, w_ref, b_ref, h32_ref, h16_ref):
    h = _bdot(x_ref[...], w_ref[...]) + b_ref[...]
    h32_ref[...] = h
    h16_ref[...] = h.astype(jnp.bfloat16)


def _step_kernel(adj_ref, hb_ref, h_ref, w_gate_ref, wc_ref, b_gate_ref,
                 nh32_ref, nh16_ref, *maybe_adj16_ref, cast_adj):
    adj = adj_ref[...].astype(jnp.bfloat16)
    if cast_adj:
        maybe_adj16_ref[0][...] = adj
    m = jnp.dot(adj, hb_ref[...], preferred_element_type=jnp.float32)
    h_new = _gru_update(h_ref[...], m, w_gate_ref[...], wc_ref[...],
                        b_gate_ref[...])
    nh32_ref[...] = h_new
    nh16_ref[...] = h_new.astype(jnp.bfloat16)


def _row_spec(bm, width):
    return pl.BlockSpec((bm, width), lambda i: (i, 0))


def _full_spec(shape):
    return pl.BlockSpec(shape, lambda i: (0, 0))


def kernel(input, adj, weight, bias, candidate_weight, update_w, update_b,
           reset_w, reset_b):
    w_gate = jnp.concatenate([update_w, reset_w], axis=1)
    b_gate = jnp.concatenate([update_b, reset_b]).reshape(1, 2 * _D)

    def h_shapes(rows):
        return [jax.ShapeDtypeStruct((rows, _D), jnp.float32),
                jax.ShapeDtypeStruct((rows, _D), jnp.bfloat16)]

    h32, h16 = pl.pallas_call(
        _h0_kernel,
        grid=(_N // 1000,),
        in_specs=[_row_spec(1000, _D), _full_spec((_D, _D)),
                  _full_spec((1, _D))],
        out_specs=[_row_spec(1000, _D), _row_spec(1000, _D)],
        out_shape=h_shapes(_N),
    )(input, weight, bias.reshape(1, _D))

    small_specs = [
        _full_spec((2 * _D, 2 * _D)), _full_spec((_D, _D)),
        _full_spec((1, 2 * _D)),
    ]
    small_args = (w_gate, candidate_weight, b_gate)

    h32, h16, adj16 = pl.pallas_call(
        functools.partial(_step_kernel, cast_adj=True),
        grid=(_N // _BM1,),
        in_specs=[_row_spec(_BM1, _N), _full_spec((_N, _D)),
                  _row_spec(_BM1, _D)] + small_specs,
        out_specs=[_row_spec(_BM1, _D), _row_spec(_BM1, _D),
                   _row_spec(_BM1, _N)],
        out_shape=h_shapes(_N) + [jax.ShapeDtypeStruct((_N, _N),
                                                       jnp.bfloat16)],
    )(adj, h16, h32, *small_args)

    step = pl.pallas_call(
        functools.partial(_step_kernel, cast_adj=False),
        grid=(_N // _BM2,),
        in_specs=[_row_spec(_BM2, _N), _full_spec((_N, _D)),
                  _row_spec(_BM2, _D)] + small_specs,
        out_specs=[_row_spec(_BM2, _D), _row_spec(_BM2, _D)],
        out_shape=h_shapes(_N),
    )
    h32, h16 = step(adj16, h16, h32, *small_args)
    h32, _ = step(adj16, h16, h32, *small_args)
    return h32


# speedup vs baseline: 2.5811x; 1.0334x over previous
import functools

import jax
import jax.numpy as jnp
from jax.experimental import pallas as pl
from jax.experimental.pallas import tpu as pltpu

_N = 10000
_D = 128
_BM1 = 400
_BM2 = 1000


def _bdot(a, b):
    return jnp.dot(a.astype(jnp.bfloat16), b.astype(jnp.bfloat16),
                   preferred_element_type=jnp.float32)


def _gru_update(h, m, w_gate, wc, b_gate):
    gi = jnp.concatenate([h, m], axis=1)
    pre = _bdot(gi, w_gate) + b_gate
    z = jax.nn.sigmoid(pre[:, :_D])
    r = jax.nn.sigmoid(pre[:, _D:])
    cand = jnp.tanh(_bdot(r * h, wc))
    return z * h + (1.0 - z) * cand


def _h0_kernel(x_ref, w_ref, b_ref, h32_ref, h16_ref):
    h = _bdot(x_ref[...], w_ref[...]) + b_ref[...]
    h32_ref[...] = h
    h16_ref[...] = h.astype(jnp.bfloat16)


def _step_kernel(adj_ref, hb_ref, h_ref, w_gate_ref, wc_ref, b_gate_ref,
                 nh32_ref, nh16_ref, *maybe_adj16_ref, cast_adj):
    adj = adj_ref[...].astype(jnp.bfloat16)
    if cast_adj:
        maybe_adj16_ref[0][...] = adj
    m = jnp.dot(adj, hb_ref[...], preferred_element_type=jnp.float32)
    h_new = _gru_update(h_ref[...], m, w_gate_ref[...], wc_ref[...],
                        b_gate_ref[...])
    nh32_ref[...] = h_new
    nh16_ref[...] = h_new.astype(jnp.bfloat16)


def _step1_kernel_pipe(adj_ref, hb_ref, h_ref, w_gate_ref, wc_ref,
                       b_gate_ref, nh32_ref, nh16_ref, adj16_ref, m_scr, *,
                       nblocks):
    i = pl.program_id(0)

    @pl.when(i < nblocks)
    def _spmm():
        adj = adj_ref[...].astype(jnp.bfloat16)
        adj16_ref[...] = adj
        m_scr[i % 2] = jnp.dot(adj, hb_ref[...],
                               preferred_element_type=jnp.float32)

    @pl.when(i > 0)
    def _gru():
        h_new = _gru_update(h_ref[...], m_scr[(i - 1) % 2], w_gate_ref[...],
                            wc_ref[...], b_gate_ref[...])
        nh32_ref[...] = h_new
        nh16_ref[...] = h_new.astype(jnp.bfloat16)


def _step_kernel_pipe(adj_ref, hb_ref, h_ref, w_gate_ref, wc_ref, b_gate_ref,
                      nh32_ref, nh16_ref, m_scr, *, nblocks):
    i = pl.program_id(0)

    @pl.when(i < nblocks)
    def _spmm():
        m_scr[i % 2] = jnp.dot(adj_ref[...], hb_ref[...],
                               preferred_element_type=jnp.float32)

    @pl.when(i > 0)
    def _gru():
        h_new = _gru_update(h_ref[...], m_scr[(i - 1) % 2], w_gate_ref[...],
                            wc_ref[...], b_gate_ref[...])
        nh32_ref[...] = h_new
        nh16_ref[...] = h_new.astype(jnp.bfloat16)


def _row_spec(bm, width):
    return pl.BlockSpec((bm, width), lambda i: (i, 0))


def _full_spec(shape):
    return pl.BlockSpec(shape, lambda i: (0, 0))


def kernel(input, adj, weight, bias, candidate_weight, update_w, update_b,
           reset_w, reset_b):
    w_gate = jnp.concatenate([update_w, reset_w], axis=1)
    b_gate = jnp.concatenate([update_b, reset_b]).reshape(1, 2 * _D)

    def h_shapes(rows):
        return [jax.ShapeDtypeStruct((rows, _D), jnp.float32),
                jax.ShapeDtypeStruct((rows, _D), jnp.bfloat16)]

    h32, h16 = pl.pallas_call(
        _h0_kernel,
        grid=(1,),
        in_specs=[_row_spec(_N, _D), _full_spec((_D, _D)),
                  _full_spec((1, _D))],
        out_specs=[_row_spec(_N, _D), _row_spec(_N, _D)],
        out_shape=h_shapes(_N),
    )(input, weight, bias.reshape(1, _D))

    small_specs = [
        _full_spec((2 * _D, 2 * _D)), _full_spec((_D, _D)),
        _full_spec((1, 2 * _D)),
    ]
    small_args = (w_gate, candidate_weight, b_gate)

    nb1 = _N // _BM1
    lag1 = pl.BlockSpec((_BM1, _D), lambda i: (jnp.maximum(i - 1, 0), 0))
    clamp1 = lambda i: (jnp.minimum(i, nb1 - 1), 0)
    h32, h16, adj16 = pl.pallas_call(
        functools.partial(_step1_kernel_pipe, nblocks=nb1),
        grid=(nb1 + 1,),
        in_specs=[pl.BlockSpec((_BM1, _N), clamp1), _full_spec((_N, _D)),
                  lag1] + small_specs,
        out_specs=[lag1, lag1, pl.BlockSpec((_BM1, _N), clamp1)],
        out_shape=h_shapes(_N) + [jax.ShapeDtypeStruct((_N, _N),
                                                       jnp.bfloat16)],
        scratch_shapes=[pltpu.VMEM((2, _BM1, _D), jnp.float32)],
    )(adj, h16, h32, *small_args)

    nb = _N // _BM2
    lag_in = pl.BlockSpec((_BM2, _D),
                          lambda i: (jnp.maximum(i - 1, 0), 0))
    lag_out = pl.BlockSpec((_BM2, _D),
                           lambda i: (jnp.maximum(i - 1, 0), 0))
    clamp_adj = pl.BlockSpec((_BM2, _N),
                             lambda i: (jnp.minimum(i, nb - 1), 0))
    step = pl.pallas_call(
        functools.partial(_step_kernel_pipe, nblocks=nb),
        grid=(nb + 1,),
        in_specs=[clamp_adj, _full_spec((_N, _D)), lag_in] + small_specs,
        out_specs=[lag_out, lag_out],
        out_shape=h_shapes(_N),
        scratch_shapes=[pltpu.VMEM((2, _BM2, _D), jnp.float32)],
    )
    h32, h16 = step(adj16, h16, h32, *small_args)
    h32, _ = step(adj16, h16, h32, *small_args)
    return h32
